# baseline (speedup 1.0000x reference)
import jax
import jax.numpy as jnp
from jax.experimental import pallas as pl
from jax.experimental.pallas import tpu as pltpu

_NS = 4


def _gcn_kernel(x_ref, adj_ref, w_ref, b_ref, o_ref, y_ref):
    S, B, H = x_ref.shape
    O = w_ref.shape[1]
    i = pl.program_id(0)
    j = pl.program_id(1)

    @pl.when((i == 0) & (j == 0))
    def _():
        xf = x_ref[...].reshape(S * B, H).astype(jnp.bfloat16)
        y = jnp.dot(xf, w_ref[...],
                    preferred_element_type=jnp.float32) + b_ref[...]
        y3 = y.astype(jnp.bfloat16).reshape(S, B, O)
        for bb in range(B):
            y_ref[bb * S:(bb + 1) * S, :] = y3[:, bb, :]

    y_b = y_ref[pl.ds(j * S, S), :]
    z = jnp.dot(adj_ref[...], y_b,
                preferred_element_type=jnp.float32)
    o_ref[:, j, :] = jnp.maximum(z, 0.0)


def kernel(x, adj, w, b):
    S, B, H = x.shape
    O = w.shape[0]
    tS = S // _NS

    adj_bm = jnp.transpose(adj.astype(jnp.bfloat16), (2, 0, 1))
    w_t = jnp.transpose(w).astype(jnp.bfloat16)
    b2d = b.reshape(1, O).astype(jnp.float32)

    return pl.pallas_call(
        _gcn_kernel,
        out_shape=jax.ShapeDtypeStruct((S, B, O), jnp.float32),
        grid_spec=pltpu.PrefetchScalarGridSpec(
            num_scalar_prefetch=0,
            grid=(_NS, B),
            in_specs=[
                pl.BlockSpec((S, B, H), lambda i, j: (0, 0, 0)),
                pl.BlockSpec((None, tS, S), lambda i, j: (j, i, 0)),
                pl.BlockSpec((H, O), lambda i, j: (0, 0)),
                pl.BlockSpec((1, O), lambda i, j: (0, 0)),
            ],
            out_specs=pl.BlockSpec((tS, B, O), lambda i, j: (i, 0, 0)),
            scratch_shapes=[pltpu.VMEM((B * S, O), jnp.bfloat16)],
        ),
        compiler_params=pltpu.CompilerParams(
            dimension_semantics=("arbitrary", "arbitrary"),
            vmem_limit_bytes=64 * 1024 * 1024,
        ),
    )(x, adj_bm, w_t, b2d)


# speedup vs baseline: 1.2385x; 1.2385x over previous
import jax
import jax.numpy as jnp
from jax.experimental import pallas as pl
from jax.experimental.pallas import tpu as pltpu

_NS = 1


def _gcn_kernel(x_ref, adj_ref, w_ref, b_ref, o_ref, y_ref):
    S, B, H = x_ref.shape
    O = w_ref.shape[1]
    i = pl.program_id(0)
    j = pl.program_id(1)

    @pl.when((i == 0) & (j == 0))
    def _():
        xf = x_ref[...].reshape(S * B, H).astype(jnp.bfloat16)
        y = jnp.dot(xf, w_ref[...],
                    preferred_element_type=jnp.float32) + b_ref[...]
        y3 = y.astype(jnp.bfloat16).reshape(S, B, O)
        for bb in range(B):
            y_ref[bb * S:(bb + 1) * S, :] = y3[:, bb, :]

    y_b = y_ref[pl.ds(j * S, S), :]
    z = jnp.dot(adj_ref[...], y_b,
                preferred_element_type=jnp.float32)
    o_ref[:, j, :] = jnp.maximum(z, 0.0)


def kernel(x, adj, w, b):
    S, B, H = x.shape
    O = w.shape[0]
    tS = S // _NS

    adj_bm = jnp.transpose(adj.astype(jnp.bfloat16), (2, 0, 1))
    w_t = jnp.transpose(w).astype(jnp.bfloat16)
    b2d = b.reshape(1, O).astype(jnp.float32)

    return pl.pallas_call(
        _gcn_kernel,
        out_shape=jax.ShapeDtypeStruct((S, B, O), jnp.float32),
        grid_spec=pltpu.PrefetchScalarGridSpec(
            num_scalar_prefetch=0,
            grid=(_NS, B),
            in_specs=[
                pl.BlockSpec((S, B, H), lambda i, j: (0, 0, 0)),
                pl.BlockSpec((None, tS, S), lambda i, j: (j, i, 0)),
                pl.BlockSpec((H, O), lambda i, j: (0, 0)),
                pl.BlockSpec((1, O), lambda i, j: (0, 0)),
            ],
            out_specs=pl.BlockSpec((tS, B, O), lambda i, j: (i, 0, 0)),
            scratch_shapes=[pltpu.VMEM((B * S, O), jnp.bfloat16)],
        ),
        compiler_params=pltpu.CompilerParams(
            dimension_semantics=("arbitrary", "arbitrary"),
            vmem_limit_bytes=64 * 1024 * 1024,
        ),
    )(x, adj_bm, w_t, b2d)
